# baseline (speedup 1.0000x reference)
import functools

import jax
import jax.numpy as jnp
from jax import lax
from jax.experimental import pallas as pl
from jax.experimental.pallas import tpu as pltpu
from jax.experimental.pallas import tpu_sc as plsc

N = 10000
E = 320000
D = 128
L = 3
G = 64

NC = 2
NS = 16
NW = NC * NS

CH = 128
C = 80
SPLIT = 4
SCH = CH // SPLIT
F_CHUNKS = 120
S_CHUNKS = 40
CT = NS * (F_CHUNKS + S_CHUNKS)
E_PAD = CT * CH
N_PAD = 10240
ZR = 64
ZROWS = N_PAD // NS
OUT_STRIDE = N_PAD // NS
OUT_LAST = N - (NS - 1) * OUT_STRIDE

_mesh = plsc.VectorSubcoreMesh(
    core_axis_name="c", subcore_axis_name="s", num_cores=NC, num_subcores=NS)


def _zero_fill(buf, rows, width):
    zero = jnp.zeros((16,), jnp.float32)
    per_row = width // 16

    def body(i, _):
        r = i // per_row
        k = lax.rem(i, per_row)
        buf[r, pl.ds(k * 16, 16)] = zero
        return 0

    lax.fori_loop(0, rows * per_row, body, 0)


@functools.partial(
    pl.kernel,
    out_type=jax.ShapeDtypeStruct((NC, N, D), jnp.float32),
    mesh=_mesh,
    scratch_types=[
        pltpu.VMEM((F_CHUNKS, CH), jnp.int32),
        pltpu.VMEM((CH,), jnp.int32),
        pltpu.VMEM((CH,), jnp.int32),
        pltpu.VMEM((CH, D), jnp.float32),
        pltpu.VMEM((CH, D), jnp.float32),
        pltpu.VMEM_SHARED((N_PAD, D), jnp.float32),
        pltpu.SemaphoreType.DMA,
        pltpu.SemaphoreType.DMA,
        pltpu.SemaphoreType.DMA,
        pltpu.SemaphoreType.DMA,
    ],
)
def _sc_agg(hn_hbm, srcp_hbm, dstp_hbm, out_hbm,
            dstv, ib0, ib1, rows0, rows1, agg_sh,
            isem0, isem1, rsem0, rsem1):
    c = lax.axis_index("c")
    s = lax.axis_index("s")

    _zero_fill(rows0, ZR, D)

    def zstep(k, _):
        pltpu.sync_copy(rows0.at[pl.ds(0, ZR)],
                        agg_sh.at[pl.ds(s * ZROWS + k * ZR, ZR)])
        return 0

    lax.fori_loop(0, ZROWS // ZR, zstep, 0)

    plsc.subcore_barrier()

    table = hn_hbm.at[c]

    def start_gather(ib, rows, rsem):
        for q in range(SPLIT):
            pltpu.async_copy(table.at[ib.at[pl.ds(q * SCH, SCH)]],
                             rows.at[pl.ds(q * SCH, SCH)], rsem)

    def wait_gather(ib, rows, rsem):
        for q in range(SPLIT):
            pltpu.make_async_copy(table.at[ib.at[pl.ds(q * SCH, SCH)]],
                                  rows.at[pl.ds(q * SCH, SCH)], rsem).wait()

    def run_pipeline(base, ncs):
        pltpu.sync_copy(dstp_hbm.at[pl.ds(base, ncs)], dstv.at[pl.ds(0, ncs)])
        pltpu.async_copy(srcp_hbm.at[base], ib0, isem0)
        pltpu.async_copy(srcp_hbm.at[base + 1], ib1, isem1)
        pltpu.make_async_copy(srcp_hbm.at[base], ib0, isem0).wait()
        start_gather(ib0, rows0, rsem0)

        def step(jj, _):
            j0 = jj * 2
            last = jj == ncs // 2 - 1

            pltpu.make_async_copy(srcp_hbm.at[base + j0 + 1], ib1, isem1).wait()
            start_gather(ib1, rows1, rsem1)

            wait_gather(ib0, rows0, rsem0)

            @pl.when(jnp.logical_not(last))
            def _():
                pltpu.async_copy(srcp_hbm.at[base + j0 + 2], ib0, isem0)

            pltpu.sync_copy(rows0, agg_sh.at[dstv.at[j0]], add=True)

            @pl.when(jnp.logical_not(last))
            def _():
                pltpu.make_async_copy(srcp_hbm.at[base + j0 + 2], ib0,
                                      isem0).wait()
                start_gather(ib0, rows0, rsem0)

            wait_gather(ib1, rows1, rsem1)

            @pl.when(jnp.logical_not(last))
            def _():
                pltpu.async_copy(srcp_hbm.at[base + j0 + 3], ib1, isem1)

            pltpu.sync_copy(rows1, agg_sh.at[dstv.at[j0 + 1]], add=True)
            return 0

        lax.fori_loop(0, ncs // 2, step, 0)

    @pl.when(c == 0)
    def _():
        run_pipeline(s * F_CHUNKS, F_CHUNKS)

    @pl.when(c == 1)
    def _():
        run_pipeline(NS * F_CHUNKS + s * S_CHUNKS, S_CHUNKS)

    plsc.subcore_barrier()

    @pl.when(s < NS - 1)
    def _():
        pltpu.sync_copy(agg_sh.at[pl.ds(s * OUT_STRIDE, OUT_STRIDE)],
                        out_hbm.at[c, pl.ds(s * OUT_STRIDE, OUT_STRIDE)])

    @pl.when(s == NS - 1)
    def _():
        pltpu.sync_copy(agg_sh.at[pl.ds((NS - 1) * OUT_STRIDE, OUT_LAST)],
                        out_hbm.at[c, pl.ds((NS - 1) * OUT_STRIDE, OUT_LAST)])


@functools.partial(
    pl.kernel,
    out_type=jax.ShapeDtypeStruct((NC, N, D), jnp.float32),
    mesh=_mesh,
    scratch_types=[
        pltpu.VMEM((C, CH), jnp.int32),
        pltpu.VMEM((CH, D), jnp.float32),
        pltpu.VMEM_SHARED((N_PAD, D), jnp.float32),
    ],
)
def _sc_deg(dstp_hbm, out_hbm, dstv, ones, deg_sh):
    c = lax.axis_index("c")
    s = lax.axis_index("s")
    wid = s * NC + c

    _zero_fill(ones, ZR, D)

    def zstep(k, _):
        pltpu.sync_copy(ones.at[pl.ds(0, ZR)],
                        deg_sh.at[pl.ds(s * ZROWS + k * ZR, ZR)])
        return 0

    lax.fori_loop(0, ZROWS // ZR, zstep, 0)

    one = jnp.ones((16,), jnp.float32)

    def ofill(i, _):
        r = i // (D // 16)
        k = lax.rem(i, D // 16)
        ones[r, pl.ds(k * 16, 16)] = one
        return 0

    lax.fori_loop(0, CH * (D // 16), ofill, 0)

    pltpu.sync_copy(dstp_hbm.at[pl.ds(wid * C, C)], dstv)
    plsc.subcore_barrier()

    def step(j, _):
        pltpu.sync_copy(ones, deg_sh.at[dstv.at[j]], add=True)
        return 0

    lax.fori_loop(0, C, step, 0)
    plsc.subcore_barrier()

    @pl.when(s < NS - 1)
    def _():
        pltpu.sync_copy(deg_sh.at[pl.ds(s * OUT_STRIDE, OUT_STRIDE)],
                        out_hbm.at[c, pl.ds(s * OUT_STRIDE, OUT_STRIDE)])

    @pl.when(s == NS - 1)
    def _():
        pltpu.sync_copy(deg_sh.at[pl.ds((NS - 1) * OUT_STRIDE, OUT_LAST)],
                        out_hbm.at[c, pl.ds((NS - 1) * OUT_STRIDE, OUT_LAST)])


def _tc_pre_body(x_ref, wn0_ref, hn0_ref):
    hn = jnp.dot(x_ref[...], wn0_ref[...], preferred_element_type=jnp.float32)
    hn0_ref[0] = hn
    hn0_ref[1] = hn


def _mean_from_parts(aggp_ref, degp_ref):
    deg = jnp.maximum(degp_ref[0, :, 0:1] + degp_ref[1, :, 0:1], 1.0)
    return (aggp_ref[0] + aggp_ref[1]) / deg


def _tc_mid_body(h_ref, aggp_ref, degp_ref, wr_ref, b_ref, wn_next_ref,
                 h1_ref, hn1_ref):
    mean = _mean_from_parts(aggp_ref, degp_ref)
    h1 = jnp.dot(h_ref[...], wr_ref[...], preferred_element_type=jnp.float32)
    h1 = jnp.maximum(h1 + mean + b_ref[0], 0.0)
    h1_ref[...] = h1
    hn1 = jnp.dot(h1, wn_next_ref[...], preferred_element_type=jnp.float32)
    hn1_ref[0] = hn1
    hn1_ref[1] = hn1


def _tc_final_body(h_ref, aggp_ref, degp_ref, wr_ref, b_ref, batch_ref,
                   out_ref):
    mean = _mean_from_parts(aggp_ref, degp_ref)
    h3 = jnp.dot(h_ref[...], wr_ref[...], preferred_element_type=jnp.float32)
    h3 = h3 + mean + b_ref[0]
    gids = lax.broadcasted_iota(jnp.int32, (G, N), 0)
    onehot = jnp.where(gids == batch_ref[...][None, :], 1.0, 0.0)
    out_ref[...] = jnp.dot(onehot, h3, preferred_element_type=jnp.float32)


_tc_pre = pl.pallas_call(
    _tc_pre_body,
    out_shape=jax.ShapeDtypeStruct((NC, N, D), jnp.float32),
)

_tc_mid = pl.pallas_call(
    _tc_mid_body,
    out_shape=(jax.ShapeDtypeStruct((N, D), jnp.float32),
               jax.ShapeDtypeStruct((NC, N, D), jnp.float32)),
)

_tc_final = pl.pallas_call(
    _tc_final_body,
    out_shape=jax.ShapeDtypeStruct((G, D), jnp.float32),
)


def kernel(x, edge_index, batch, W_root, W_neigh, b):
    src = edge_index[0]
    dst = edge_index[1]
    pad = E_PAD - E
    srcp = jnp.concatenate([src, jnp.zeros((pad,), jnp.int32)]).reshape(CT, CH)
    dstp = jnp.concatenate([dst, jnp.full((pad,), N, jnp.int32)]).reshape(CT, CH)

    degp = _sc_deg(dstp)
    hn = _tc_pre(x, W_neigh[0])

    h = x
    for i in range(L - 1):
        aggp = _sc_agg(hn, srcp, dstp)
        h, hn = _tc_mid(h, aggp, degp, W_root[i], b[i:i + 1], W_neigh[i + 1])

    aggp = _sc_agg(hn, srcp, dstp)
    return _tc_final(h, aggp, degp, W_root[L - 1], b[L - 1:L], batch)


# speedup vs baseline: 3.1209x; 3.1209x over previous
import functools

import jax
import jax.numpy as jnp
from jax import lax
from jax.experimental import pallas as pl
from jax.experimental.pallas import tpu as pltpu
from jax.experimental.pallas import tpu_sc as plsc

N = 10000
E = 320000
D = 128
L = 3
G = 64

NC = 2
NS = 16
NW = NC * NS

CH = 128
C = 80
SPLIT = 4
SCH = CH // SPLIT
F_CHUNKS = 80
S_CHUNKS = 80
CT = NS * (F_CHUNKS + S_CHUNKS)
E_PAD = CT * CH
N_PAD = 10240
ZR = 64
ZROWS = N_PAD // NS
OUT_STRIDE = N_PAD // NS
OUT_LAST = N - (NS - 1) * OUT_STRIDE

_mesh = plsc.VectorSubcoreMesh(
    core_axis_name="c", subcore_axis_name="s", num_cores=NC, num_subcores=NS)


def _zero_fill(buf, rows, width):
    zero = jnp.zeros((16,), jnp.float32)
    per_row = width // 16

    def body(i, _):
        r = i // per_row
        k = lax.rem(i, per_row)
        buf[r, pl.ds(k * 16, 16)] = zero
        return 0

    lax.fori_loop(0, rows * per_row, body, 0)


@functools.partial(
    pl.kernel,
    out_type=jax.ShapeDtypeStruct((NC, N, D), jnp.float32),
    mesh=_mesh,
    scratch_types=[
        pltpu.VMEM((F_CHUNKS, CH), jnp.int32),
        pltpu.VMEM((CH,), jnp.int32),
        pltpu.VMEM((CH,), jnp.int32),
        pltpu.VMEM((CH, D), jnp.float32),
        pltpu.VMEM((CH, D), jnp.float32),
        pltpu.VMEM_SHARED((N_PAD, D), jnp.float32),
        pltpu.SemaphoreType.DMA,
        pltpu.SemaphoreType.DMA,
        pltpu.SemaphoreType.DMA,
        pltpu.SemaphoreType.DMA,
    ],
)
def _sc_agg(hn_hbm, srcp_hbm, dstp_hbm, out_hbm,
            dstv, ib0, ib1, rows0, rows1, agg_sh,
            isem0, isem1, rsem0, rsem1):
    c = lax.axis_index("c")
    s = lax.axis_index("s")

    _zero_fill(rows0, ZR, D)

    def zstep(k, _):
        pltpu.sync_copy(rows0.at[pl.ds(0, ZR)],
                        agg_sh.at[pl.ds(s * ZROWS + k * ZR, ZR)])
        return 0

    lax.fori_loop(0, ZROWS // ZR, zstep, 0)

    plsc.subcore_barrier()

    table = hn_hbm.at[c]

    def start_gather(ib, rows, rsem):
        for q in range(SPLIT):
            pltpu.async_copy(table.at[ib.at[pl.ds(q * SCH, SCH)]],
                             rows.at[pl.ds(q * SCH, SCH)], rsem)

    def wait_gather(ib, rows, rsem):
        for q in range(SPLIT):
            pltpu.make_async_copy(table.at[ib.at[pl.ds(q * SCH, SCH)]],
                                  rows.at[pl.ds(q * SCH, SCH)], rsem).wait()

    def run_pipeline(base, ncs):
        pltpu.sync_copy(dstp_hbm.at[pl.ds(base, ncs)], dstv.at[pl.ds(0, ncs)])
        pltpu.async_copy(srcp_hbm.at[base], ib0, isem0)
        pltpu.async_copy(srcp_hbm.at[base + 1], ib1, isem1)
        pltpu.make_async_copy(srcp_hbm.at[base], ib0, isem0).wait()
        start_gather(ib0, rows0, rsem0)

        def step(jj, _):
            j0 = jj * 2
            last = jj == ncs // 2 - 1

            pltpu.make_async_copy(srcp_hbm.at[base + j0 + 1], ib1, isem1).wait()
            start_gather(ib1, rows1, rsem1)

            wait_gather(ib0, rows0, rsem0)

            @pl.when(jnp.logical_not(last))
            def _():
                pltpu.async_copy(srcp_hbm.at[base + j0 + 2], ib0, isem0)

            pltpu.sync_copy(rows0, agg_sh.at[dstv.at[j0]], add=True)

            @pl.when(jnp.logical_not(last))
            def _():
                pltpu.make_async_copy(srcp_hbm.at[base + j0 + 2], ib0,
                                      isem0).wait()
                start_gather(ib0, rows0, rsem0)

            wait_gather(ib1, rows1, rsem1)

            @pl.when(jnp.logical_not(last))
            def _():
                pltpu.async_copy(srcp_hbm.at[base + j0 + 3], ib1, isem1)

            pltpu.sync_copy(rows1, agg_sh.at[dstv.at[j0 + 1]], add=True)
            return 0

        lax.fori_loop(0, ncs // 2, step, 0)

    @pl.when(c == 0)
    def _():
        run_pipeline(s * F_CHUNKS, F_CHUNKS)

    @pl.when(c == 1)
    def _():
        run_pipeline(NS * F_CHUNKS + s * S_CHUNKS, S_CHUNKS)

    plsc.subcore_barrier()

    @pl.when(s < NS - 1)
    def _():
        pltpu.sync_copy(agg_sh.at[pl.ds(s * OUT_STRIDE, OUT_STRIDE)],
                        out_hbm.at[c, pl.ds(s * OUT_STRIDE, OUT_STRIDE)])

    @pl.when(s == NS - 1)
    def _():
        pltpu.sync_copy(agg_sh.at[pl.ds((NS - 1) * OUT_STRIDE, OUT_LAST)],
                        out_hbm.at[c, pl.ds((NS - 1) * OUT_STRIDE, OUT_LAST)])


@functools.partial(
    pl.kernel,
    out_type=jax.ShapeDtypeStruct((NC, N, D), jnp.float32),
    mesh=_mesh,
    scratch_types=[
        pltpu.VMEM((C, CH), jnp.int32),
        pltpu.VMEM((CH, D), jnp.float32),
        pltpu.VMEM_SHARED((N_PAD, D), jnp.float32),
    ],
)
def _sc_deg(dstp_hbm, out_hbm, dstv, ones, deg_sh):
    c = lax.axis_index("c")
    s = lax.axis_index("s")
    wid = s * NC + c

    _zero_fill(ones, ZR, D)

    def zstep(k, _):
        pltpu.sync_copy(ones.at[pl.ds(0, ZR)],
                        deg_sh.at[pl.ds(s * ZROWS + k * ZR, ZR)])
        return 0

    lax.fori_loop(0, ZROWS // ZR, zstep, 0)

    one = jnp.ones((16,), jnp.float32)

    def ofill(i, _):
        r = i // (D // 16)
        k = lax.rem(i, D // 16)
        ones[r, pl.ds(k * 16, 16)] = one
        return 0

    lax.fori_loop(0, CH * (D // 16), ofill, 0)

    pltpu.sync_copy(dstp_hbm.at[pl.ds(wid * C, C)], dstv)
    plsc.subcore_barrier()

    def step(j, _):
        pltpu.sync_copy(ones, deg_sh.at[dstv.at[j]], add=True)
        return 0

    lax.fori_loop(0, C, step, 0)
    plsc.subcore_barrier()

    @pl.when(s < NS - 1)
    def _():
        pltpu.sync_copy(deg_sh.at[pl.ds(s * OUT_STRIDE, OUT_STRIDE)],
                        out_hbm.at[c, pl.ds(s * OUT_STRIDE, OUT_STRIDE)])

    @pl.when(s == NS - 1)
    def _():
        pltpu.sync_copy(deg_sh.at[pl.ds((NS - 1) * OUT_STRIDE, OUT_LAST)],
                        out_hbm.at[c, pl.ds((NS - 1) * OUT_STRIDE, OUT_LAST)])


def _tc_pre_body(x_ref, wn0_ref, hn0_ref):
    hn = jnp.dot(x_ref[...], wn0_ref[...], preferred_element_type=jnp.float32)
    hn0_ref[0] = hn
    hn0_ref[1] = hn


def _mean_from_parts(aggp_ref, degp_ref):
    deg = jnp.maximum(degp_ref[0, :, 0:1] + degp_ref[1, :, 0:1], 1.0)
    return (aggp_ref[0] + aggp_ref[1]) / deg


def _tc_mid_body(h_ref, aggp_ref, degp_ref, wr_ref, b_ref, wn_next_ref,
                 h1_ref, hn1_ref):
    mean = _mean_from_parts(aggp_ref, degp_ref)
    h1 = jnp.dot(h_ref[...], wr_ref[...], preferred_element_type=jnp.float32)
    h1 = jnp.maximum(h1 + mean + b_ref[0], 0.0)
    h1_ref[...] = h1
    hn1 = jnp.dot(h1, wn_next_ref[...], preferred_element_type=jnp.float32)
    hn1_ref[0] = hn1
    hn1_ref[1] = hn1


def _tc_final_body(h_ref, aggp_ref, degp_ref, wr_ref, b_ref, batch_ref,
                   out_ref):
    mean = _mean_from_parts(aggp_ref, degp_ref)
    h3 = jnp.dot(h_ref[...], wr_ref[...], preferred_element_type=jnp.float32)
    h3 = h3 + mean + b_ref[0]
    gids = lax.broadcasted_iota(jnp.int32, (G, N), 0)
    onehot = jnp.where(gids == batch_ref[...][None, :], 1.0, 0.0)
    out_ref[...] = jnp.dot(onehot, h3, preferred_element_type=jnp.float32)


_tc_pre = pl.pallas_call(
    _tc_pre_body,
    out_shape=jax.ShapeDtypeStruct((NC, N, D), jnp.float32),
)

_tc_mid = pl.pallas_call(
    _tc_mid_body,
    out_shape=(jax.ShapeDtypeStruct((N, D), jnp.float32),
               jax.ShapeDtypeStruct((NC, N, D), jnp.float32)),
)

_tc_final = pl.pallas_call(
    _tc_final_body,
    out_shape=jax.ShapeDtypeStruct((G, D), jnp.float32),
)


def kernel(x, edge_index, batch, W_root, W_neigh, b):
    src = edge_index[0]
    dst = edge_index[1]
    pad = E_PAD - E
    pad_src = (jnp.arange(pad, dtype=jnp.int32) * 37) % N
    srcp = jnp.concatenate([src, pad_src]).reshape(CT, CH)
    dstp = jnp.concatenate([dst, jnp.full((pad,), N, jnp.int32)]).reshape(CT, CH)

    degp = _sc_deg(dstp)
    hn = _tc_pre(x, W_neigh[0])

    h = x
    for i in range(L - 1):
        aggp = _sc_agg(hn, srcp, dstp)
        h, hn = _tc_mid(h, aggp, degp, W_root[i], b[i:i + 1], W_neigh[i + 1])

    aggp = _sc_agg(hn, srcp, dstp)
    return _tc_final(h, aggp, degp, W_root[L - 1], b[L - 1:L], batch)


# speedup vs baseline: 3.1544x; 1.0107x over previous
import functools

import jax
import jax.numpy as jnp
from jax import lax
from jax.experimental import pallas as pl
from jax.experimental.pallas import tpu as pltpu
from jax.experimental.pallas import tpu_sc as plsc

N = 10000
E = 320000
D = 128
L = 3
G = 64

NC = 2
NS = 16
NW = NC * NS

CH = 128
C = 80
SPLIT = 4
SCH = CH // SPLIT
F_CHUNKS = 80
S_CHUNKS = 80
CT = NS * (F_CHUNKS + S_CHUNKS)
E_PAD = CT * CH
N_PAD = 10240
ZR = 64
DEG_R = N_PAD // D
ZROWS = N_PAD // NS
OUT_STRIDE = N_PAD // NS
OUT_LAST = N - (NS - 1) * OUT_STRIDE

_mesh = plsc.VectorSubcoreMesh(
    core_axis_name="c", subcore_axis_name="s", num_cores=NC, num_subcores=NS)


def _zero_fill(buf, rows, width):
    zero = jnp.zeros((16,), jnp.float32)
    per_row = width // 16

    def body(i, _):
        r = i // per_row
        k = lax.rem(i, per_row)
        buf[r, pl.ds(k * 16, 16)] = zero
        return 0

    lax.fori_loop(0, rows * per_row, body, 0)


@functools.partial(
    pl.kernel,
    out_type=jax.ShapeDtypeStruct((NC, N, D), jnp.float32),
    mesh=_mesh,
    scratch_types=[
        pltpu.VMEM((F_CHUNKS, CH), jnp.int32),
        pltpu.VMEM((CH,), jnp.int32),
        pltpu.VMEM((CH,), jnp.int32),
        pltpu.VMEM((CH, D), jnp.float32),
        pltpu.VMEM((CH, D), jnp.float32),
        pltpu.VMEM_SHARED((N_PAD, D), jnp.float32),
        pltpu.SemaphoreType.DMA,
        pltpu.SemaphoreType.DMA,
        pltpu.SemaphoreType.DMA,
        pltpu.SemaphoreType.DMA,
    ],
)
def _sc_agg(hn_hbm, srcp_hbm, dstp_hbm, out_hbm,
            dstv, ib0, ib1, rows0, rows1, agg_sh,
            isem0, isem1, rsem0, rsem1):
    c = lax.axis_index("c")
    s = lax.axis_index("s")

    _zero_fill(rows0, ZR, D)

    def zstep(k, _):
        pltpu.sync_copy(rows0.at[pl.ds(0, ZR)],
                        agg_sh.at[pl.ds(s * ZROWS + k * ZR, ZR)])
        return 0

    lax.fori_loop(0, ZROWS // ZR, zstep, 0)

    plsc.subcore_barrier()

    table = hn_hbm

    def start_gather(ib, rows, rsem):
        for q in range(SPLIT):
            pltpu.async_copy(table.at[ib.at[pl.ds(q * SCH, SCH)]],
                             rows.at[pl.ds(q * SCH, SCH)], rsem)

    def wait_gather(ib, rows, rsem):
        for q in range(SPLIT):
            pltpu.make_async_copy(table.at[ib.at[pl.ds(q * SCH, SCH)]],
                                  rows.at[pl.ds(q * SCH, SCH)], rsem).wait()

    def run_pipeline(base, ncs):
        pltpu.sync_copy(dstp_hbm.at[pl.ds(base, ncs)], dstv.at[pl.ds(0, ncs)])
        pltpu.async_copy(srcp_hbm.at[base], ib0, isem0)
        pltpu.async_copy(srcp_hbm.at[base + 1], ib1, isem1)
        pltpu.make_async_copy(srcp_hbm.at[base], ib0, isem0).wait()
        start_gather(ib0, rows0, rsem0)

        def step(jj, _):
            j0 = jj * 2
            last = jj == ncs // 2 - 1

            pltpu.make_async_copy(srcp_hbm.at[base + j0 + 1], ib1, isem1).wait()
            start_gather(ib1, rows1, rsem1)

            wait_gather(ib0, rows0, rsem0)

            @pl.when(jnp.logical_not(last))
            def _():
                pltpu.async_copy(srcp_hbm.at[base + j0 + 2], ib0, isem0)

            pltpu.sync_copy(rows0, agg_sh.at[dstv.at[j0]], add=True)

            @pl.when(jnp.logical_not(last))
            def _():
                pltpu.make_async_copy(srcp_hbm.at[base + j0 + 2], ib0,
                                      isem0).wait()
                start_gather(ib0, rows0, rsem0)

            wait_gather(ib1, rows1, rsem1)

            @pl.when(jnp.logical_not(last))
            def _():
                pltpu.async_copy(srcp_hbm.at[base + j0 + 3], ib1, isem1)

            pltpu.sync_copy(rows1, agg_sh.at[dstv.at[j0 + 1]], add=True)
            return 0

        lax.fori_loop(0, ncs // 2, step, 0)

    @pl.when(c == 0)
    def _():
        run_pipeline(s * F_CHUNKS, F_CHUNKS)

    @pl.when(c == 1)
    def _():
        run_pipeline(NS * F_CHUNKS + s * S_CHUNKS, S_CHUNKS)

    plsc.subcore_barrier()

    @pl.when(s < NS - 1)
    def _():
        pltpu.sync_copy(agg_sh.at[pl.ds(s * OUT_STRIDE, OUT_STRIDE)],
                        out_hbm.at[c, pl.ds(s * OUT_STRIDE, OUT_STRIDE)])

    @pl.when(s == NS - 1)
    def _():
        pltpu.sync_copy(agg_sh.at[pl.ds((NS - 1) * OUT_STRIDE, OUT_LAST)],
                        out_hbm.at[c, pl.ds((NS - 1) * OUT_STRIDE, OUT_LAST)])


@functools.partial(
    pl.kernel,
    out_type=jax.ShapeDtypeStruct((NC, N, D), jnp.float32),
    mesh=_mesh,
    scratch_types=[
        pltpu.VMEM((C, CH), jnp.int32),
        pltpu.VMEM((CH, D), jnp.float32),
        pltpu.VMEM_SHARED((N_PAD, D), jnp.float32),
    ],
)
def _sc_deg(dstp_hbm, out_hbm, dstv, ones, deg_sh):
    c = lax.axis_index("c")
    s = lax.axis_index("s")
    wid = s * NC + c

    _zero_fill(ones, ZR, D)

    def zstep(k, _):
        pltpu.sync_copy(ones.at[pl.ds(0, ZR)],
                        deg_sh.at[pl.ds(s * ZROWS + k * ZR, ZR)])
        return 0

    lax.fori_loop(0, ZROWS // ZR, zstep, 0)

    one = jnp.ones((16,), jnp.float32)

    def ofill(i, _):
        r = i // (D // 16)
        k = lax.rem(i, D // 16)
        ones[r, pl.ds(k * 16, 16)] = one
        return 0

    lax.fori_loop(0, CH * (D // 16), ofill, 0)

    pltpu.sync_copy(dstp_hbm.at[pl.ds(wid * C, C)], dstv)
    plsc.subcore_barrier()

    def step(j, _):
        pltpu.sync_copy(ones, deg_sh.at[dstv.at[j]], add=True)
        return 0

    lax.fori_loop(0, C, step, 0)
    plsc.subcore_barrier()

    @pl.when(s < NS - 1)
    def _():
        pltpu.sync_copy(deg_sh.at[pl.ds(s * OUT_STRIDE, OUT_STRIDE)],
                        out_hbm.at[c, pl.ds(s * OUT_STRIDE, OUT_STRIDE)])

    @pl.when(s == NS - 1)
    def _():
        pltpu.sync_copy(deg_sh.at[pl.ds((NS - 1) * OUT_STRIDE, OUT_LAST)],
                        out_hbm.at[c, pl.ds((NS - 1) * OUT_STRIDE, OUT_LAST)])


def _tc_pre_body(x_ref, wn0_ref, hn0_ref):
    hn0_ref[...] = jnp.dot(x_ref[...], wn0_ref[...],
                           preferred_element_type=jnp.float32)


def _mean_from_parts(aggp_ref, degp_ref):
    deg = jnp.maximum(degp_ref[0, :, 0:1] + degp_ref[1, :, 0:1], 1.0)
    return (aggp_ref[0] + aggp_ref[1]) / deg


def _tc_mid_body(h_ref, aggp_ref, degp_ref, wr_ref, b_ref, wn_next_ref,
                 h1_ref, hn1_ref):
    mean = _mean_from_parts(aggp_ref, degp_ref)
    h1 = jnp.dot(h_ref[...], wr_ref[...], preferred_element_type=jnp.float32)
    h1 = jnp.maximum(h1 + mean + b_ref[0], 0.0)
    h1_ref[...] = h1
    hn1_ref[...] = jnp.dot(h1, wn_next_ref[...],
                           preferred_element_type=jnp.float32)


def _tc_final_body(h_ref, aggp_ref, degp_ref, wr_ref, b_ref, batch_ref,
                   out_ref):
    mean = _mean_from_parts(aggp_ref, degp_ref)
    h3 = jnp.dot(h_ref[...], wr_ref[...], preferred_element_type=jnp.float32)
    h3 = h3 + mean + b_ref[0]
    gids = lax.broadcasted_iota(jnp.int32, (G, N), 0)
    onehot = jnp.where(gids == batch_ref[...][None, :], 1.0, 0.0)
    out_ref[...] = jnp.dot(onehot, h3, preferred_element_type=jnp.float32)


_tc_pre = pl.pallas_call(
    _tc_pre_body,
    out_shape=jax.ShapeDtypeStruct((N, D), jnp.float32),
)

_tc_mid = pl.pallas_call(
    _tc_mid_body,
    out_shape=(jax.ShapeDtypeStruct((N, D), jnp.float32),
               jax.ShapeDtypeStruct((N, D), jnp.float32)),
)

_tc_final = pl.pallas_call(
    _tc_final_body,
    out_shape=jax.ShapeDtypeStruct((G, D), jnp.float32),
)


def kernel(x, edge_index, batch, W_root, W_neigh, b):
    src = edge_index[0]
    dst = edge_index[1]
    pad = E_PAD - E
    pad_src = (jnp.arange(pad, dtype=jnp.int32) * 37) % N
    srcp = jnp.concatenate([src, pad_src]).reshape(CT, CH)
    dstp = jnp.concatenate([dst, jnp.full((pad,), N, jnp.int32)]).reshape(CT, CH)

    degp = _sc_deg(dstp)
    hn = _tc_pre(x, W_neigh[0])

    h = x
    for i in range(L - 1):
        aggp = _sc_agg(hn, srcp, dstp)
        h, hn = _tc_mid(h, aggp, degp, W_root[i], b[i:i + 1], W_neigh[i + 1])

    aggp = _sc_agg(hn, srcp, dstp)
    return _tc_final(h, aggp, degp, W_root[L - 1], b[L - 1:L], batch)


# speedup vs baseline: 3.1593x; 1.0015x over previous
import functools

import jax
import jax.numpy as jnp
from jax import lax
from jax.experimental import pallas as pl
from jax.experimental.pallas import tpu as pltpu
from jax.experimental.pallas import tpu_sc as plsc

N = 10000
E = 320000
D = 128
L = 3
G = 64

NC = 2
NS = 16
NW = NC * NS

CH = 128
C = 80
SPLIT = 8
SCH = CH // SPLIT
F_CHUNKS = 80
S_CHUNKS = 80
CT = NS * (F_CHUNKS + S_CHUNKS)
E_PAD = CT * CH
N_PAD = 10240
ZR = 64
DEG_R = N_PAD // D
ZROWS = N_PAD // NS
OUT_STRIDE = N_PAD // NS
OUT_LAST = N - (NS - 1) * OUT_STRIDE

_mesh = plsc.VectorSubcoreMesh(
    core_axis_name="c", subcore_axis_name="s", num_cores=NC, num_subcores=NS)


def _zero_fill(buf, rows, width):
    zero = jnp.zeros((16,), jnp.float32)
    per_row = width // 16

    def body(i, _):
        r = i // per_row
        k = lax.rem(i, per_row)
        buf[r, pl.ds(k * 16, 16)] = zero
        return 0

    lax.fori_loop(0, rows * per_row, body, 0)


@functools.partial(
    pl.kernel,
    out_type=jax.ShapeDtypeStruct((NC, N, D), jnp.float32),
    mesh=_mesh,
    scratch_types=[
        pltpu.VMEM((F_CHUNKS, CH), jnp.int32),
        pltpu.VMEM((CH,), jnp.int32),
        pltpu.VMEM((CH,), jnp.int32),
        pltpu.VMEM((CH, D), jnp.float32),
        pltpu.VMEM((CH, D), jnp.float32),
        pltpu.VMEM_SHARED((N_PAD, D), jnp.float32),
        pltpu.SemaphoreType.DMA,
        pltpu.SemaphoreType.DMA,
        pltpu.SemaphoreType.DMA,
        pltpu.SemaphoreType.DMA,
    ],
)
def _sc_agg(hn_hbm, srcp_hbm, dstp_hbm, out_hbm,
            dstv, ib0, ib1, rows0, rows1, agg_sh,
            isem0, isem1, rsem0, rsem1):
    c = lax.axis_index("c")
    s = lax.axis_index("s")

    _zero_fill(rows0, ZR, D)

    def zstep(k, _):
        pltpu.sync_copy(rows0.at[pl.ds(0, ZR)],
                        agg_sh.at[pl.ds(s * ZROWS + k * ZR, ZR)])
        return 0

    lax.fori_loop(0, ZROWS // ZR, zstep, 0)

    plsc.subcore_barrier()

    table = hn_hbm

    def start_gather(ib, rows, rsem):
        for q in range(SPLIT):
            pltpu.async_copy(table.at[ib.at[pl.ds(q * SCH, SCH)]],
                             rows.at[pl.ds(q * SCH, SCH)], rsem)

    def wait_gather(ib, rows, rsem):
        for q in range(SPLIT):
            pltpu.make_async_copy(table.at[ib.at[pl.ds(q * SCH, SCH)]],
                                  rows.at[pl.ds(q * SCH, SCH)], rsem).wait()

    def run_pipeline(base, ncs):
        pltpu.sync_copy(dstp_hbm.at[pl.ds(base, ncs)], dstv.at[pl.ds(0, ncs)])
        pltpu.async_copy(srcp_hbm.at[base], ib0, isem0)
        pltpu.async_copy(srcp_hbm.at[base + 1], ib1, isem1)
        pltpu.make_async_copy(srcp_hbm.at[base], ib0, isem0).wait()
        start_gather(ib0, rows0, rsem0)

        def step(jj, _):
            j0 = jj * 2
            last = jj == ncs // 2 - 1

            pltpu.make_async_copy(srcp_hbm.at[base + j0 + 1], ib1, isem1).wait()
            start_gather(ib1, rows1, rsem1)

            wait_gather(ib0, rows0, rsem0)

            @pl.when(jnp.logical_not(last))
            def _():
                pltpu.async_copy(srcp_hbm.at[base + j0 + 2], ib0, isem0)

            pltpu.sync_copy(rows0, agg_sh.at[dstv.at[j0]], add=True)

            @pl.when(jnp.logical_not(last))
            def _():
                pltpu.make_async_copy(srcp_hbm.at[base + j0 + 2], ib0,
                                      isem0).wait()
                start_gather(ib0, rows0, rsem0)

            wait_gather(ib1, rows1, rsem1)

            @pl.when(jnp.logical_not(last))
            def _():
                pltpu.async_copy(srcp_hbm.at[base + j0 + 3], ib1, isem1)

            pltpu.sync_copy(rows1, agg_sh.at[dstv.at[j0 + 1]], add=True)
            return 0

        lax.fori_loop(0, ncs // 2, step, 0)

    @pl.when(c == 0)
    def _():
        run_pipeline(s * F_CHUNKS, F_CHUNKS)

    @pl.when(c == 1)
    def _():
        run_pipeline(NS * F_CHUNKS + s * S_CHUNKS, S_CHUNKS)

    plsc.subcore_barrier()

    @pl.when(s < NS - 1)
    def _():
        pltpu.sync_copy(agg_sh.at[pl.ds(s * OUT_STRIDE, OUT_STRIDE)],
                        out_hbm.at[c, pl.ds(s * OUT_STRIDE, OUT_STRIDE)])

    @pl.when(s == NS - 1)
    def _():
        pltpu.sync_copy(agg_sh.at[pl.ds((NS - 1) * OUT_STRIDE, OUT_LAST)],
                        out_hbm.at[c, pl.ds((NS - 1) * OUT_STRIDE, OUT_LAST)])


@functools.partial(
    pl.kernel,
    out_type=jax.ShapeDtypeStruct((NC, N, D), jnp.float32),
    mesh=_mesh,
    scratch_types=[
        pltpu.VMEM((C, CH), jnp.int32),
        pltpu.VMEM((CH, D), jnp.float32),
        pltpu.VMEM_SHARED((N_PAD, D), jnp.float32),
    ],
)
def _sc_deg(dstp_hbm, out_hbm, dstv, ones, deg_sh):
    c = lax.axis_index("c")
    s = lax.axis_index("s")
    wid = s * NC + c

    _zero_fill(ones, ZR, D)

    def zstep(k, _):
        pltpu.sync_copy(ones.at[pl.ds(0, ZR)],
                        deg_sh.at[pl.ds(s * ZROWS + k * ZR, ZR)])
        return 0

    lax.fori_loop(0, ZROWS // ZR, zstep, 0)

    one = jnp.ones((16,), jnp.float32)

    def ofill(i, _):
        r = i // (D // 16)
        k = lax.rem(i, D // 16)
        ones[r, pl.ds(k * 16, 16)] = one
        return 0

    lax.fori_loop(0, CH * (D // 16), ofill, 0)

    pltpu.sync_copy(dstp_hbm.at[pl.ds(wid * C, C)], dstv)
    plsc.subcore_barrier()

    def step(j, _):
        pltpu.sync_copy(ones, deg_sh.at[dstv.at[j]], add=True)
        return 0

    lax.fori_loop(0, C, step, 0)
    plsc.subcore_barrier()

    @pl.when(s < NS - 1)
    def _():
        pltpu.sync_copy(deg_sh.at[pl.ds(s * OUT_STRIDE, OUT_STRIDE)],
                        out_hbm.at[c, pl.ds(s * OUT_STRIDE, OUT_STRIDE)])

    @pl.when(s == NS - 1)
    def _():
        pltpu.sync_copy(deg_sh.at[pl.ds((NS - 1) * OUT_STRIDE, OUT_LAST)],
                        out_hbm.at[c, pl.ds((NS - 1) * OUT_STRIDE, OUT_LAST)])


def _tc_pre_body(x_ref, wn0_ref, hn0_ref):
    hn0_ref[...] = jnp.dot(x_ref[...], wn0_ref[...],
                           preferred_element_type=jnp.float32)


def _mean_from_parts(aggp_ref, degp_ref):
    deg = jnp.maximum(degp_ref[0, :, 0:1] + degp_ref[1, :, 0:1], 1.0)
    return (aggp_ref[0] + aggp_ref[1]) / deg


def _tc_mid_body(h_ref, aggp_ref, degp_ref, wr_ref, b_ref, wn_next_ref,
                 h1_ref, hn1_ref):
    mean = _mean_from_parts(aggp_ref, degp_ref)
    h1 = jnp.dot(h_ref[...], wr_ref[...], preferred_element_type=jnp.float32)
    h1 = jnp.maximum(h1 + mean + b_ref[0], 0.0)
    h1_ref[...] = h1
    hn1_ref[...] = jnp.dot(h1, wn_next_ref[...],
                           preferred_element_type=jnp.float32)


def _tc_final_body(h_ref, aggp_ref, degp_ref, wr_ref, b_ref, batch_ref,
                   out_ref):
    mean = _mean_from_parts(aggp_ref, degp_ref)
    h3 = jnp.dot(h_ref[...], wr_ref[...], preferred_element_type=jnp.float32)
    h3 = h3 + mean + b_ref[0]
    gids = lax.broadcasted_iota(jnp.int32, (G, N), 0)
    onehot = jnp.where(gids == batch_ref[...][None, :], 1.0, 0.0)
    out_ref[...] = jnp.dot(onehot, h3, preferred_element_type=jnp.float32)


_tc_pre = pl.pallas_call(
    _tc_pre_body,
    out_shape=jax.ShapeDtypeStruct((N, D), jnp.float32),
)

_tc_mid = pl.pallas_call(
    _tc_mid_body,
    out_shape=(jax.ShapeDtypeStruct((N, D), jnp.float32),
               jax.ShapeDtypeStruct((N, D), jnp.float32)),
)

_tc_final = pl.pallas_call(
    _tc_final_body,
    out_shape=jax.ShapeDtypeStruct((G, D), jnp.float32),
)


def kernel(x, edge_index, batch, W_root, W_neigh, b):
    src = edge_index[0]
    dst = edge_index[1]
    pad = E_PAD - E
    pad_src = (jnp.arange(pad, dtype=jnp.int32) * 37) % N
    srcp = jnp.concatenate([src, pad_src]).reshape(CT, CH)
    dstp = jnp.concatenate([dst, jnp.full((pad,), N, jnp.int32)]).reshape(CT, CH)

    degp = _sc_deg(dstp)
    hn = _tc_pre(x, W_neigh[0])

    h = x
    for i in range(L - 1):
        aggp = _sc_agg(hn, srcp, dstp)
        h, hn = _tc_mid(h, aggp, degp, W_root[i], b[i:i + 1], W_neigh[i + 1])

    aggp = _sc_agg(hn, srcp, dstp)
    return _tc_final(h, aggp, degp, W_root[L - 1], b[L - 1:L], batch)
